# baseline (speedup 1.0000x reference)
import functools

import jax
import jax.numpy as jnp
from jax import lax
from jax.experimental import pallas as pl
from jax.experimental.pallas import tpu as pltpu
from jax.experimental.pallas import tpu_sc as plsc

_CH = 128
_NW = 32
_K = 1
_G = 3
_EDGE_QUANT = _NW * _CH * _K * _G


def _prop_partials(ha, hb, src2d, dst2d, zeros, ident):
    n_pad, f = ha.shape
    cpw = src2d.shape[0] // _NW
    nb = cpw // _K
    nr = n_pad // 16
    nri = nr // _CH
    npad_rows = (_G - 1) * _K
    two_in = hb is not None

    mesh = plsc.VectorSubcoreMesh(core_axis_name="c", subcore_axis_name="s")

    @functools.partial(
        pl.kernel,
        out_type=(
            jax.ShapeDtypeStruct((n_pad, f), jnp.float32),
            jax.ShapeDtypeStruct((n_pad, f), jnp.float32),
        ),
        mesh=mesh,
        scratch_types=[
            pltpu.VMEM((cpw + npad_rows, _CH), jnp.int32),
            pltpu.VMEM((cpw, _CH), jnp.int32),
            pltpu.VMEM((nri, _CH), jnp.int32),
            [[pltpu.VMEM((_CH, f), jnp.float32)] * _K] * _G,
            pltpu.VMEM_SHARED((n_pad, f), jnp.float32),
            pltpu.VMEM_SHARED((n_pad, f), jnp.float32),
            [pltpu.SemaphoreType.DMA] * _G,
            [pltpu.SemaphoreType.DMA] * _G,
        ],
        compiler_params=pltpu.CompilerParams(use_tc_tiling_on_sc=False),
    )
    def prop(ha_hbm, hb_hbm, src_hbm, dst_hbm, z_hbm, id_hbm, out0, out1,
             si_all, di_all, ii, groups, acc, h_spm, sg, ss):
        cid = lax.axis_index("c")
        sid = lax.axis_index("s")
        wid = sid * 2 + cid
        base = wid * cpw
        r0 = sid * nr

        pltpu.sync_copy(src_hbm.at[pl.ds(base, cpw)], si_all.at[pl.ds(0, cpw)])
        pltpu.sync_copy(src_hbm.at[pl.ds(0, npad_rows)],
                        si_all.at[pl.ds(cpw, npad_rows)])
        pltpu.sync_copy(dst_hbm.at[pl.ds(base, cpw)], di_all)

        def stage(dst_sh):
            pltpu.sync_copy(ha_hbm.at[pl.ds(r0, nr)], dst_sh.at[pl.ds(r0, nr)])
            if two_in:
                for c in range(nri):
                    pltpu.sync_copy(hb_hbm.at[pl.ds(r0 + c * _CH, _CH)],
                                    groups[0][0])
                    pltpu.sync_copy(groups[0][0], dst_sh.at[ii.at[c]],
                                    add=True)

        if two_in:
            pltpu.sync_copy(id_hbm.at[pl.ds(sid * nri, nri)], ii)
        stage(h_spm)

        @pl.when(cid == 0)
        def _():
            stage(acc)

        @pl.when(cid == 1)
        def _():
            pltpu.sync_copy(z_hbm.at[pl.ds(r0, nr)], acc.at[pl.ds(r0, nr)])

        plsc.subcore_barrier()

        def gathers(b, g):
            for k in range(_K):
                idx = si_all.at[b * _K + k]
                if two_in:
                    pltpu.async_copy(h_spm.at[idx], groups[g][k], sg[g])
                else:
                    hbm_src = (b % 4) == 0

                    @pl.when(hbm_src)
                    def _():
                        pltpu.async_copy(ha_hbm.at[idx], groups[g][k], sg[g])

                    @pl.when(jnp.logical_not(hbm_src))
                    def _():
                        pltpu.async_copy(h_spm.at[idx], groups[g][k], sg[g])

        def wait_gathers(b, g):
            for k in range(_K):
                pltpu.make_async_copy(h_spm.at[si_all.at[b * _K + k]],
                                      groups[g][k], sg[g]).wait()

        def scatters(b, g):
            for k in range(_K):
                pltpu.async_copy(groups[g][k], acc.at[di_all.at[b * _K + k]],
                                 ss[g], add=True)

        def wait_scatters(b, g):
            for k in range(_K):
                pltpu.make_async_copy(groups[g][k],
                                      acc.at[di_all.at[b * _K + k]],
                                      ss[g]).wait()

        for g in range(_G - 1):
            gathers(g, g)

        @pl.loop(0, nb, step=_G)
        def _(b0):
            for i in range(_G):
                wait_gathers(b0 + i, i)
                gathers(b0 + i + _G - 1, (i - 1) % _G)
                scatters(b0 + i, i)
                wait_scatters(b0 + i, i)

        for g in range(_G - 1):
            wait_gathers(nb + g, g)
        plsc.subcore_barrier()

        @pl.when(cid == 0)
        def _():
            pltpu.sync_copy(acc.at[pl.ds(r0, nr)], out0.at[pl.ds(r0, nr)])

        @pl.when(cid == 1)
        def _():
            pltpu.sync_copy(acc.at[pl.ds(r0, nr)], out1.at[pl.ds(r0, nr)])

    hb_arg = hb if two_in else ha
    return prop(ha, hb_arg, src2d, dst2d, zeros, ident)


def _matmul_tc(x, w):
    n_pad = x.shape[0]
    f = w.shape[1]

    def body(x_ref, w_ref, o_ref):
        o_ref[...] = jnp.dot(x_ref[...], w_ref[...],
                             preferred_element_type=jnp.float32)

    return pl.pallas_call(
        body,
        out_shape=jax.ShapeDtypeStruct((n_pad, f), jnp.float32),
    )(x, w)


def _dense_tc(p0, p1, gamma, beta, w2, n):
    n_pad, f = p0.shape
    c = w2.shape[1]
    scale = 1.0507009873554804934193349852946
    alpha = 1.6732632423543772848170429916717

    def body(a_ref, b_ref, g_ref, be_ref, w_ref, o_ref):
        h2 = a_ref[...] + b_ref[...]
        rows = lax.broadcasted_iota(jnp.int32, (n_pad, 1), 0)
        mask = (rows < n).astype(jnp.float32)
        hm = h2 * mask
        mean = jnp.sum(hm, axis=0, keepdims=True) / n
        var = jnp.sum(hm * hm, axis=0, keepdims=True) / n - mean * mean
        xb = (h2 - mean) * lax.rsqrt(var + 1e-5) * g_ref[...] + be_ref[...]
        s = scale * jnp.where(xb > 0, xb, alpha * (jnp.exp(xb) - 1.0))
        o_ref[...] = jnp.dot(s, w_ref[...], preferred_element_type=jnp.float32)

    return pl.pallas_call(
        body,
        out_shape=jax.ShapeDtypeStruct((n_pad, c), jnp.float32),
    )(p0, p1, gamma.reshape(1, f), beta.reshape(1, f), w2)


def _final_tc(p0, p1, b2, n):
    c = p0.shape[1]

    def body(a_ref, b_ref, bias_ref, o_ref):
        y = a_ref[...] + b_ref[...] + bias_ref[...]
        y = y[:n]
        m = jnp.max(y, axis=1, keepdims=True)
        lse = jnp.log(jnp.sum(jnp.exp(y - m), axis=1, keepdims=True)) + m
        o_ref[...] = y - lse

    return pl.pallas_call(
        body,
        out_shape=jax.ShapeDtypeStruct((n, c), jnp.float32),
    )(p0, p1, b2.reshape(1, c))


def kernel(x, edge_index, W1, b1, gamma, beta, W2, b2):
    n, d = x.shape
    e = edge_index.shape[1]
    h_dim = W1.shape[1]
    c_dim = W2.shape[1]

    n_pad = ((n + 1 + 2047) // 2048) * 2048
    e_pad = -(-e // _EDGE_QUANT) * _EDGE_QUANT

    src = edge_index[0].astype(jnp.int32)
    dst = edge_index[1].astype(jnp.int32)
    pad_idx = jnp.full((e_pad - e,), n, jnp.int32)
    src_p = jnp.concatenate([src, pad_idx]).reshape(e_pad // _CH, _CH)
    dst_p = jnp.concatenate([dst, pad_idx]).reshape(e_pad // _CH, _CH)
    x_p = jnp.zeros((n_pad, d), jnp.float32).at[:n].set(x)

    z_h = jnp.zeros((n_pad, h_dim), jnp.float32)
    z_c = jnp.zeros((n_pad, c_dim), jnp.float32)
    ident = jnp.arange(n_pad, dtype=jnp.int32).reshape(n_pad // _CH, _CH)

    h0 = _matmul_tc(x_p, W1)
    a0, a1 = _prop_partials(h0, None, src_p, dst_p, z_h, ident)
    b0_, b1_ = _prop_partials(a0, a1, src_p, dst_p, z_h, ident)
    g = _dense_tc(b0_, b1_, gamma, beta, W2, n)
    c0_, c1_ = _prop_partials(g, None, src_p, dst_p, z_c, ident)
    return _final_tc(c0_, c1_, b2, n)


# speedup vs baseline: 1.2746x; 1.2746x over previous
import functools

import jax
import jax.numpy as jnp
from jax import lax
from jax.experimental import pallas as pl
from jax.experimental.pallas import tpu as pltpu
from jax.experimental.pallas import tpu_sc as plsc

_CH = 128
_NW = 32
_K = 1
_G = 3
_EDGE_QUANT = _NW * _CH * _K * _G


def _prop_partials(ha, hb, src2d, dst2d, zeros, ident):
    n_pad, f = ha.shape
    cpw = src2d.shape[0] // _NW
    nb = cpw // _K
    nr = n_pad // 16
    nri = nr // _CH
    npad_rows = (_G - 1) * _K
    two_in = hb is not None

    mesh = plsc.VectorSubcoreMesh(core_axis_name="c", subcore_axis_name="s")

    @functools.partial(
        pl.kernel,
        out_type=(
            jax.ShapeDtypeStruct((n_pad, f), jnp.float32),
            jax.ShapeDtypeStruct((n_pad, f), jnp.float32),
        ),
        mesh=mesh,
        scratch_types=[
            pltpu.VMEM((cpw + npad_rows, _CH), jnp.int32),
            pltpu.VMEM((cpw, _CH), jnp.int32),
            pltpu.VMEM((nri, _CH), jnp.int32),
            [[pltpu.VMEM((_CH, f), jnp.float32)] * _K] * _G,
            pltpu.VMEM_SHARED((n_pad, f), jnp.float32),
            pltpu.VMEM_SHARED((n_pad, f), jnp.float32),
            [pltpu.SemaphoreType.DMA] * _G,
            [pltpu.SemaphoreType.DMA] * _G,
        ],
        compiler_params=pltpu.CompilerParams(use_tc_tiling_on_sc=False),
    )
    def prop(ha_hbm, hb_hbm, src_hbm, dst_hbm, z_hbm, id_hbm, out0, out1,
             si_all, di_all, ii, groups, acc, h_spm, sg, ss):
        cid = lax.axis_index("c")
        sid = lax.axis_index("s")
        wid = sid * 2 + cid
        base = wid * cpw
        r0 = sid * nr

        pltpu.sync_copy(src_hbm.at[pl.ds(base, cpw)], si_all.at[pl.ds(0, cpw)])
        pltpu.sync_copy(src_hbm.at[pl.ds(0, npad_rows)],
                        si_all.at[pl.ds(cpw, npad_rows)])
        pltpu.sync_copy(dst_hbm.at[pl.ds(base, cpw)], di_all)

        def stage(dst_sh):
            pltpu.sync_copy(ha_hbm.at[pl.ds(r0, nr)], dst_sh.at[pl.ds(r0, nr)])
            if two_in:
                for c in range(nri):
                    pltpu.sync_copy(hb_hbm.at[pl.ds(r0 + c * _CH, _CH)],
                                    groups[0][0])
                    pltpu.sync_copy(groups[0][0], dst_sh.at[ii.at[c]],
                                    add=True)

        if two_in:
            pltpu.sync_copy(id_hbm.at[pl.ds(sid * nri, nri)], ii)
        stage(h_spm)

        @pl.when(cid == 0)
        def _():
            pltpu.sync_copy(ha_hbm.at[pl.ds(r0, nr)], acc.at[pl.ds(r0, nr)])

        @pl.when(cid == 1)
        def _():
            other = hb_hbm if two_in else z_hbm
            pltpu.sync_copy(other.at[pl.ds(r0, nr)], acc.at[pl.ds(r0, nr)])

        plsc.subcore_barrier()

        def gathers(b, g):
            for k in range(_K):
                pltpu.async_copy(h_spm.at[si_all.at[b * _K + k]],
                                 groups[g][k], sg[g])

        def wait_gathers(b, g):
            for k in range(_K):
                pltpu.make_async_copy(h_spm.at[si_all.at[b * _K + k]],
                                      groups[g][k], sg[g]).wait()

        def scatters(b, g):
            for k in range(_K):
                pltpu.async_copy(groups[g][k], acc.at[di_all.at[b * _K + k]],
                                 ss[g], add=True)

        def wait_scatters(b, g):
            for k in range(_K):
                pltpu.make_async_copy(groups[g][k],
                                      acc.at[di_all.at[b * _K + k]],
                                      ss[g]).wait()

        for g in range(_G - 1):
            gathers(g, g)

        @pl.loop(0, nb, step=_G)
        def _(b0):
            for i in range(_G):
                wait_gathers(b0 + i, i)
                gathers(b0 + i + _G - 1, (i - 1) % _G)
                scatters(b0 + i, i)
                wait_scatters(b0 + i, i)

        for g in range(_G - 1):
            wait_gathers(nb + g, g)
        plsc.subcore_barrier()

        @pl.when(cid == 0)
        def _():
            pltpu.sync_copy(acc.at[pl.ds(r0, nr)], out0.at[pl.ds(r0, nr)])

        @pl.when(cid == 1)
        def _():
            pltpu.sync_copy(acc.at[pl.ds(r0, nr)], out1.at[pl.ds(r0, nr)])

    hb_arg = hb if two_in else ha
    return prop(ha, hb_arg, src2d, dst2d, zeros, ident)


def _matmul_tc(x, w):
    n_pad = x.shape[0]
    f = w.shape[1]

    def body(x_ref, w_ref, o_ref):
        o_ref[...] = jnp.dot(x_ref[...], w_ref[...],
                             preferred_element_type=jnp.float32)

    return pl.pallas_call(
        body,
        out_shape=jax.ShapeDtypeStruct((n_pad, f), jnp.float32),
    )(x, w)


def _dense_tc(p0, p1, gamma, beta, w2, n):
    n_pad, f = p0.shape
    c = w2.shape[1]
    scale = 1.0507009873554804934193349852946
    alpha = 1.6732632423543772848170429916717

    def body(a_ref, b_ref, g_ref, be_ref, w_ref, o_ref):
        h2 = a_ref[...] + b_ref[...]
        rows = lax.broadcasted_iota(jnp.int32, (n_pad, 1), 0)
        mask = (rows < n).astype(jnp.float32)
        hm = h2 * mask
        mean = jnp.sum(hm, axis=0, keepdims=True) / n
        var = jnp.sum(hm * hm, axis=0, keepdims=True) / n - mean * mean
        xb = (h2 - mean) * lax.rsqrt(var + 1e-5) * g_ref[...] + be_ref[...]
        s = scale * jnp.where(xb > 0, xb, alpha * (jnp.exp(xb) - 1.0))
        o_ref[...] = jnp.dot(s, w_ref[...], preferred_element_type=jnp.float32)

    return pl.pallas_call(
        body,
        out_shape=jax.ShapeDtypeStruct((n_pad, c), jnp.float32),
    )(p0, p1, gamma.reshape(1, f), beta.reshape(1, f), w2)


def _final_tc(p0, p1, b2, n):
    c = p0.shape[1]

    def body(a_ref, b_ref, bias_ref, o_ref):
        y = a_ref[...] + b_ref[...] + bias_ref[...]
        y = y[:n]
        m = jnp.max(y, axis=1, keepdims=True)
        lse = jnp.log(jnp.sum(jnp.exp(y - m), axis=1, keepdims=True)) + m
        o_ref[...] = y - lse

    return pl.pallas_call(
        body,
        out_shape=jax.ShapeDtypeStruct((n, c), jnp.float32),
    )(p0, p1, b2.reshape(1, c))


def kernel(x, edge_index, W1, b1, gamma, beta, W2, b2):
    n, d = x.shape
    e = edge_index.shape[1]
    h_dim = W1.shape[1]
    c_dim = W2.shape[1]

    n_pad = ((n + 1 + 2047) // 2048) * 2048
    e_pad = -(-e // _EDGE_QUANT) * _EDGE_QUANT

    src = edge_index[0].astype(jnp.int32)
    dst = edge_index[1].astype(jnp.int32)
    pad_idx = jnp.full((e_pad - e,), n, jnp.int32)
    src_p = jnp.concatenate([src, pad_idx]).reshape(e_pad // _CH, _CH)
    dst_p = jnp.concatenate([dst, pad_idx]).reshape(e_pad // _CH, _CH)
    x_p = jnp.zeros((n_pad, d), jnp.float32).at[:n].set(x)

    z_h = jnp.zeros((n_pad, h_dim), jnp.float32)
    z_c = jnp.zeros((n_pad, c_dim), jnp.float32)
    ident = jnp.arange(n_pad, dtype=jnp.int32).reshape(n_pad // _CH, _CH)

    h0 = _matmul_tc(x_p, W1)
    a0, a1 = _prop_partials(h0, None, src_p, dst_p, z_h, ident)
    b0_, b1_ = _prop_partials(a0, a1, src_p, dst_p, z_h, ident)
    g = _dense_tc(b0_, b1_, gamma, beta, W2, n)
    c0_, c1_ = _prop_partials(g, None, src_p, dst_p, z_c, ident)
    return _final_tc(c0_, c1_, b2, n)
